# baseline (speedup 1.0000x reference)
import math

import jax
import jax.numpy as jnp
from jax import lax
from jax.experimental import pallas as pl
from jax.experimental.pallas import tpu as pltpu
from jax.experimental.pallas import tpu_sc as plsc

B, L = 4096, 50
N = B * L
OUT_D = 63
G_D = 48
NC, NS = 2, 16
NW = NC * NS
ROWS_PER_W = B // NW
R_TILE = 8
E_TILE = R_TILE * L
N_TILES = ROWS_PER_W // R_TILE
G_CHUNKS = [(lo, min(128, E_TILE - lo)) for lo in range(0, E_TILE, 128)]


def _body(tpad_h, idx_h, enh_h, edi_h, seal_h, seg_h, suit_h, rank_h,
          scal_h, enht_h, edit_h, sealt_h, segt_h, sin_h, cos_h,
          out_h,
          out_t, g48, sp48, idx_v, enh_v, edi_v, seal_v, seg_v,
          suit_v, rank_v, scal_v, enh_t, edi_t, seal_t, seg_t,
          sin_t, cos_t, rhist, shist, sem_in, sem_g, sem_out):
    wid = lax.axis_index("s") * NC + lax.axis_index("c")
    sid = lax.axis_index("s")
    pltpu.sync_copy(enht_h, enh_t)
    pltpu.sync_copy(edit_h, edi_t)
    pltpu.sync_copy(sealt_h, seal_t)
    pltpu.sync_copy(segt_h, seg_t)
    pltpu.sync_copy(sin_h, sin_t)
    pltpu.sync_copy(cos_h, cos_t)

    iota = lax.iota(jnp.int32, 16)
    ones = jnp.ones((16,), jnp.float32)
    base_w = wid * (ROWS_PER_W * L)
    rowb_w = wid * ROWS_PER_W

    def col(c):
        return jnp.full((16,), c, jnp.int32)

    def fire_in(t, b):
        base = base_w + t * E_TILE
        rowb = rowb_w + t * R_TILE
        pltpu.async_copy(idx_h.at[pl.ds(base, E_TILE)], idx_v.at[b], sem_in)
        pltpu.async_copy(enh_h.at[pl.ds(base, E_TILE)], enh_v.at[b], sem_in)
        pltpu.async_copy(edi_h.at[pl.ds(base, E_TILE)], edi_v.at[b], sem_in)
        pltpu.async_copy(seal_h.at[pl.ds(base, E_TILE)], seal_v.at[b], sem_in)
        pltpu.async_copy(seg_h.at[pl.ds(base, E_TILE)], seg_v.at[b], sem_in)
        pltpu.async_copy(suit_h.at[pl.ds(base, E_TILE)], suit_v.at[b], sem_in)
        pltpu.async_copy(rank_h.at[pl.ds(base, E_TILE)], rank_v.at[b], sem_in)
        pltpu.async_copy(scal_h.at[pl.ds(base * 4, E_TILE * 4)], scal_v.at[b],
                         sem_in)

    def drain_in():
        pltpu.make_async_copy(idx_h.at[pl.ds(0, E_TILE)], idx_v.at[0],
                              sem_in).wait()
        pltpu.make_async_copy(enh_h.at[pl.ds(0, E_TILE)], enh_v.at[0],
                              sem_in).wait()
        pltpu.make_async_copy(edi_h.at[pl.ds(0, E_TILE)], edi_v.at[0],
                              sem_in).wait()
        pltpu.make_async_copy(seal_h.at[pl.ds(0, E_TILE)], seal_v.at[0],
                              sem_in).wait()
        pltpu.make_async_copy(seg_h.at[pl.ds(0, E_TILE)], seg_v.at[0],
                              sem_in).wait()
        pltpu.make_async_copy(suit_h.at[pl.ds(0, E_TILE)], suit_v.at[0],
                              sem_in).wait()
        pltpu.make_async_copy(rank_h.at[pl.ds(0, E_TILE)], rank_v.at[0],
                              sem_in).wait()
        pltpu.make_async_copy(scal_h.at[pl.ds(0, E_TILE * 4)], scal_v.at[0],
                              sem_in).wait()

    def fire_g(b):
        for lo, n in G_CHUNKS:
            pltpu.async_copy(tpad_h.at[idx_v.at[b, pl.ds(lo, n)]],
                             g48.at[b, pl.ds(lo, n)], sem_g)

    def drain_g():
        for lo, n in G_CHUNKS:
            pltpu.make_async_copy(tpad_h.at[pl.ds(0, n)],
                                  g48.at[0, pl.ds(0, n)], sem_g).wait()

    def fire_out(t, b):
        rowb = rowb_w + t * R_TILE
        for r in range(R_TILE):
            pltpu.async_copy(out_t.at[b, pl.ds(L * r, L)],
                             out_h.at[rowb + r], sem_out)


    def drain_out():
        for r in range(R_TILE):
            pltpu.make_async_copy(out_h.at[0], out_t.at[0, pl.ds(0, L)],
                                  sem_out).wait()


    def assemble(b):
        bb = jnp.full((16,), b, jnp.int32)
        for r in range(R_TILE):
            e0 = r * L
            rr = col(r)
            rhist[...] = jnp.zeros((16,), jnp.float32)
            shist[...] = jnp.zeros((16,), jnp.float32)
            groups = []
            for g in range(4):
                rem = min(16, L - 16 * g)
                valid = iota < rem
                lidx = jnp.minimum(16 * g + iota, L - 1)
                eidx = e0 + lidx
                rv = plsc.load_gather(rank_v, [bb, eidx])
                rv = jnp.where(valid, rv, 15)
                sv = plsc.load_gather(suit_v, [bb, eidx])
                sv = jnp.where(valid, sv, 15)
                groups.append((lidx, eidx, valid, rv, sv))
                plsc.addupdate_scatter(rhist, [rv], ones, mask=valid)
                plsc.addupdate_scatter(shist, [sv], ones, mask=valid)
            for lidx, eidx, valid, rv, sv in groups:
                same_rank = plsc.load_gather(rhist, [rv])
                same_suit = plsc.load_gather(shist, [sv])
                up = plsc.load_gather(rhist, [jnp.maximum(rv - 1, 0)])
                down = plsc.load_gather(rhist, [jnp.minimum(rv + 1, 15)])
                m0 = rv == 0
                zero = jnp.zeros((16,), jnp.float32)
                f_sr = jnp.where(m0, zero, same_rank * 0.2)
                f_ss = jnp.where(m0, zero, same_suit * 0.2)
                f_up = jnp.where(m0, zero, up)
                f_dn = jnp.where(m0, zero, down)
                sinv = plsc.load_gather(sin_t, [rv])
                cosv = plsc.load_gather(cos_t, [rv])
                plsc.addupdate_scatter(out_t, [bb, eidx, col(37)], cosv,
                                       mask=valid)
                plsc.addupdate_scatter(out_t, [bb, eidx, col(38)], sinv,
                                       mask=valid)
                plsc.addupdate_scatter(out_t, [bb, eidx, col(39)], f_dn,
                                       mask=valid)
                plsc.addupdate_scatter(out_t, [bb, eidx, col(40)], f_up,
                                       mask=valid)
                plsc.addupdate_scatter(out_t, [bb, eidx, col(41)], f_ss,
                                       mask=valid)
                plsc.addupdate_scatter(out_t, [bb, eidx, col(42)], f_sr,
                                       mask=valid)
                plsc.addupdate_scatter(out_t, [bb, eidx, sv], ones,
                                       mask=valid)
                plsc.addupdate_scatter(out_t, [bb, eidx, rv + 5], ones,
                                       mask=valid)
                for vals_ref, tab_ref, c0 in ((seg_v, seg_t, 47),
                                              (enh_v, enh_t, 51),
                                              (edi_v, edi_t, 55),
                                              (seal_v, seal_t, 59)):
                    fv = plsc.load_gather(vals_ref, [bb, eidx])
                    for c in range(4):
                        plsc.store_scatter(
                            out_t, [bb, eidx, col(c0 + c)],
                            plsc.load_gather(tab_ref, [fv, col(c)]),
                            mask=valid)
                for c in range(4):
                    s = plsc.load_gather(scal_v, [bb, eidx * 4 + c])
                    plsc.store_scatter(out_t, [bb, eidx, col(43 + c)], s,
                                       mask=valid)

    fire_in(0, 0)
    drain_in()
    fire_g(0)
    fire_in(1, 1)

    def tile_body(t, carry):
        b = t % 2
        drain_g()
        pltpu.sync_copy(g48.at[b], sp48.at[sid])
        pltpu.sync_copy(sp48.at[sid], out_t.at[b, :, pl.ds(0, G_D)])

        @pl.when(t < N_TILES - 1)
        def _():
            drain_in()

        assemble(b)

        @pl.when(t > 0)
        def _():
            drain_out()
        fire_out(t, b)

        @pl.when(t < N_TILES - 1)
        def _():
            fire_g(1 - b)

        @pl.when(t < N_TILES - 2)
        def _():
            fire_in(jnp.minimum(t + 2, N_TILES - 1), b)
        return carry

    lax.fori_loop(0, N_TILES, tile_body, 0)
    drain_out()


@jax.jit
def _run(tpad, idx_f, enh_f, edi_f, seal_f, seg_f, suit_f, rank_f,
         scal_f, enh_table, edi_table, seal_table, seg_table, sin_tab,
         cos_tab):
    mesh = plsc.VectorSubcoreMesh(core_axis_name="c", subcore_axis_name="s")
    f = pl.kernel(
        _body,
        out_type=jax.ShapeDtypeStruct((B, L, OUT_D), jnp.float32),
        mesh=mesh,
        compiler_params=pltpu.CompilerParams(needs_layout_passes=False,
                                             use_tc_tiling_on_sc=False),
        scratch_types=[
            pltpu.VMEM((2, E_TILE, OUT_D), jnp.float32),
            pltpu.VMEM((2, E_TILE, G_D), jnp.float32),
            pltpu.VMEM_SHARED((NS, E_TILE, G_D), jnp.float32),
            pltpu.VMEM((2, E_TILE), jnp.int32),
            pltpu.VMEM((2, E_TILE), jnp.int32),
            pltpu.VMEM((2, E_TILE), jnp.int32),
            pltpu.VMEM((2, E_TILE), jnp.int32),
            pltpu.VMEM((2, E_TILE), jnp.int32),
            pltpu.VMEM((2, E_TILE), jnp.int32),
            pltpu.VMEM((2, E_TILE), jnp.int32),
            pltpu.VMEM((2, E_TILE * 4), jnp.float32),
            pltpu.VMEM((16, 4), jnp.float32),
            pltpu.VMEM((8, 4), jnp.float32),
            pltpu.VMEM((8, 4), jnp.float32),
            pltpu.VMEM((16, 4), jnp.float32),
            pltpu.VMEM((16,), jnp.float32),
            pltpu.VMEM((16,), jnp.float32),
            pltpu.VMEM((16,), jnp.float32),
            pltpu.VMEM((16,), jnp.float32),
            pltpu.SemaphoreType.DMA,
            pltpu.SemaphoreType.DMA,
            pltpu.SemaphoreType.DMA,
        ],
    )
    return f(tpad, idx_f, enh_f, edi_f, seal_f, seg_f, suit_f, rank_f,
             scal_f, enh_table, edi_table, seal_table, seg_table,
             sin_tab, cos_tab)


def kernel(indices, enhancement, edition, seal, segment, suit, rank,
           scalar_properties, debuffed,
           idx_table, enh_table, edi_table, seal_table, seg_table,
           suit_table, rank_table):
    tpad = jnp.pad(idx_table, ((0, 0), (0, G_D - idx_table.shape[1])))
    ang = jnp.arange(16, dtype=jnp.float32) * (2.0 * math.pi / 13.0)
    sin_tab = jnp.sin(ang)
    cos_tab = jnp.cos(ang)
    scale = jnp.array([10.0, 100.0, 100.0, 10.0], jnp.float32)
    scal_pre = (scalar_properties
                * (1.0 - debuffed.astype(jnp.float32))[:, :, None] / scale)
    emb = _run(
        tpad,
        indices.reshape(-1), enhancement.reshape(-1), edition.reshape(-1),
        seal.reshape(-1), segment.reshape(-1), suit.reshape(-1),
        rank.reshape(-1),
        scal_pre.reshape(-1),
        enh_table, edi_table, seal_table, seg_table, sin_tab, cos_tab)
    return emb, (indices == 0) & (rank == 0)


# speedup vs baseline: 1.1099x; 1.1099x over previous
import math

import jax
import jax.numpy as jnp
from jax import lax
from jax.experimental import pallas as pl
from jax.experimental.pallas import tpu as pltpu
from jax.experimental.pallas import tpu_sc as plsc

B, L = 4096, 50
N = B * L
OUT_D = 63
G_D = 48
NC, NS = 2, 16
NW = NC * NS
ROWS_PER_W = B // NW
R_TILE = 8
E_TILE = R_TILE * L
N_TILES = ROWS_PER_W // R_TILE
G_CHUNKS = [(lo, min(128, E_TILE - lo)) for lo in range(0, E_TILE, 128)]


def _body(tpad_h, idx_h, enh_h, edi_h, seal_h, seg_h, suit_h, rank_h,
          scal_h, enht_h, edit_h, sealt_h, segt_h, sin_h, cos_h,
          out_h,
          out_t, g48, sp48, idx_v, enh_v, edi_v, seal_v, seg_v,
          suit_v, rank_v, scal_v, enh_t, edi_t, seal_t, seg_t,
          sin_t, cos_t, rhist, shist, sem_in, sem_g, sem_out):
    wid = lax.axis_index("s") * NC + lax.axis_index("c")
    sid = lax.axis_index("s")
    pltpu.sync_copy(enht_h, enh_t)
    pltpu.sync_copy(edit_h, edi_t)
    pltpu.sync_copy(sealt_h, seal_t)
    pltpu.sync_copy(segt_h, seg_t)
    pltpu.sync_copy(sin_h, sin_t)
    pltpu.sync_copy(cos_h, cos_t)

    iota = lax.iota(jnp.int32, 16)
    ones = jnp.ones((16,), jnp.float32)
    base_w = wid * (ROWS_PER_W * L)
    rowb_w = wid * ROWS_PER_W

    def col(c):
        return jnp.full((16,), c, jnp.int32)

    def fire_in(t, b):
        base = base_w + t * E_TILE
        rowb = rowb_w + t * R_TILE
        pltpu.async_copy(idx_h.at[pl.ds(base, E_TILE)], idx_v.at[b], sem_in)
        pltpu.async_copy(enh_h.at[pl.ds(base, E_TILE)], enh_v.at[b], sem_in)
        pltpu.async_copy(edi_h.at[pl.ds(base, E_TILE)], edi_v.at[b], sem_in)
        pltpu.async_copy(seal_h.at[pl.ds(base, E_TILE)], seal_v.at[b], sem_in)
        pltpu.async_copy(seg_h.at[pl.ds(base, E_TILE)], seg_v.at[b], sem_in)
        pltpu.async_copy(suit_h.at[pl.ds(base, E_TILE)], suit_v.at[b], sem_in)
        pltpu.async_copy(rank_h.at[pl.ds(base, E_TILE)], rank_v.at[b], sem_in)
        pltpu.async_copy(scal_h.at[pl.ds(base * 4, E_TILE * 4)], scal_v.at[b],
                         sem_in)

    def drain_in():
        pltpu.make_async_copy(idx_h.at[pl.ds(0, E_TILE)], idx_v.at[0],
                              sem_in).wait()
        pltpu.make_async_copy(enh_h.at[pl.ds(0, E_TILE)], enh_v.at[0],
                              sem_in).wait()
        pltpu.make_async_copy(edi_h.at[pl.ds(0, E_TILE)], edi_v.at[0],
                              sem_in).wait()
        pltpu.make_async_copy(seal_h.at[pl.ds(0, E_TILE)], seal_v.at[0],
                              sem_in).wait()
        pltpu.make_async_copy(seg_h.at[pl.ds(0, E_TILE)], seg_v.at[0],
                              sem_in).wait()
        pltpu.make_async_copy(suit_h.at[pl.ds(0, E_TILE)], suit_v.at[0],
                              sem_in).wait()
        pltpu.make_async_copy(rank_h.at[pl.ds(0, E_TILE)], rank_v.at[0],
                              sem_in).wait()
        pltpu.make_async_copy(scal_h.at[pl.ds(0, E_TILE * 4)], scal_v.at[0],
                              sem_in).wait()

    def fire_g(b):
        for lo, n in G_CHUNKS:
            pltpu.async_copy(tpad_h.at[idx_v.at[b, pl.ds(lo, n)]],
                             g48.at[b, pl.ds(lo, n)], sem_g)

    def drain_g():
        for lo, n in G_CHUNKS:
            pltpu.make_async_copy(tpad_h.at[pl.ds(0, n)],
                                  g48.at[0, pl.ds(0, n)], sem_g).wait()

    def fire_out(t, b):
        rowb = rowb_w + t * R_TILE
        for r in range(R_TILE):
            pltpu.async_copy(out_t.at[b, pl.ds(L * r, L)],
                             out_h.at[rowb + r], sem_out)


    def drain_out():
        for r in range(R_TILE):
            pltpu.make_async_copy(out_h.at[0], out_t.at[0, pl.ds(0, L)],
                                  sem_out).wait()


    def assemble(b):
        bb = jnp.full((16,), b, jnp.int32)

        @plsc.parallel_loop(0, R_TILE, step=1, carry=jnp.int32(0))
        def _rows(r, cry):
            e0 = r * L
            rr = iota * 0 + r
            zf = jnp.zeros((16,), jnp.float32)
            plsc.store_scatter(rhist, [rr, iota], zf)
            plsc.store_scatter(shist, [rr, iota], zf)
            groups = []
            for g in range(4):
                rem = min(16, L - 16 * g)
                valid = iota < rem
                lidx = jnp.minimum(16 * g + iota, L - 1)
                eidx = e0 + lidx
                rv = plsc.load_gather(rank_v, [bb, eidx])
                rv = jnp.where(valid, rv, 15)
                sv = plsc.load_gather(suit_v, [bb, eidx])
                sv = jnp.where(valid, sv, 15)
                groups.append((lidx, eidx, valid, rv, sv))
                plsc.addupdate_scatter(rhist, [rr, rv], ones, mask=valid)
                plsc.addupdate_scatter(shist, [rr, sv], ones, mask=valid)
            for lidx, eidx, valid, rv, sv in groups:
                same_rank = plsc.load_gather(rhist, [rr, rv])
                same_suit = plsc.load_gather(shist, [rr, sv])
                up = plsc.load_gather(rhist, [rr, jnp.maximum(rv - 1, 0)])
                down = plsc.load_gather(rhist, [rr, jnp.minimum(rv + 1, 15)])
                m0 = rv == 0
                zero = jnp.zeros((16,), jnp.float32)
                f_sr = jnp.where(m0, zero, same_rank * 0.2)
                f_ss = jnp.where(m0, zero, same_suit * 0.2)
                f_up = jnp.where(m0, zero, up)
                f_dn = jnp.where(m0, zero, down)
                sinv = plsc.load_gather(sin_t, [rv])
                cosv = plsc.load_gather(cos_t, [rv])
                plsc.addupdate_scatter(out_t, [bb, eidx, col(37)], cosv,
                                       mask=valid)
                plsc.addupdate_scatter(out_t, [bb, eidx, col(38)], sinv,
                                       mask=valid)
                plsc.addupdate_scatter(out_t, [bb, eidx, col(39)], f_dn,
                                       mask=valid)
                plsc.addupdate_scatter(out_t, [bb, eidx, col(40)], f_up,
                                       mask=valid)
                plsc.addupdate_scatter(out_t, [bb, eidx, col(41)], f_ss,
                                       mask=valid)
                plsc.addupdate_scatter(out_t, [bb, eidx, col(42)], f_sr,
                                       mask=valid)
                plsc.addupdate_scatter(out_t, [bb, eidx, sv], ones,
                                       mask=valid)
                plsc.addupdate_scatter(out_t, [bb, eidx, rv + 5], ones,
                                       mask=valid)
                for vals_ref, tab_ref, c0 in ((seg_v, seg_t, 47),
                                              (enh_v, enh_t, 51),
                                              (edi_v, edi_t, 55),
                                              (seal_v, seal_t, 59)):
                    fv = plsc.load_gather(vals_ref, [bb, eidx])
                    for c in range(4):
                        plsc.store_scatter(
                            out_t, [bb, eidx, col(c0 + c)],
                            plsc.load_gather(tab_ref, [fv, col(c)]),
                            mask=valid)
                for c in range(4):
                    s = plsc.load_gather(scal_v, [bb, eidx * 4 + c])
                    plsc.store_scatter(out_t, [bb, eidx, col(43 + c)], s,
                                       mask=valid)
            return cry

    fire_in(0, 0)
    drain_in()
    fire_g(0)
    fire_in(1, 1)

    def tile_body(t, carry):
        b = t % 2
        drain_g()
        pltpu.sync_copy(g48.at[b], sp48.at[sid])
        pltpu.sync_copy(sp48.at[sid], out_t.at[b, :, pl.ds(0, G_D)])

        @pl.when(t < N_TILES - 1)
        def _():
            drain_in()

        assemble(b)

        @pl.when(t > 0)
        def _():
            drain_out()
        fire_out(t, b)

        @pl.when(t < N_TILES - 1)
        def _():
            fire_g(1 - b)

        @pl.when(t < N_TILES - 2)
        def _():
            fire_in(jnp.minimum(t + 2, N_TILES - 1), b)
        return carry

    lax.fori_loop(0, N_TILES, tile_body, 0)
    drain_out()


@jax.jit
def _run(tpad, idx_f, enh_f, edi_f, seal_f, seg_f, suit_f, rank_f,
         scal_f, enh_table, edi_table, seal_table, seg_table, sin_tab,
         cos_tab):
    mesh = plsc.VectorSubcoreMesh(core_axis_name="c", subcore_axis_name="s")
    f = pl.kernel(
        _body,
        out_type=jax.ShapeDtypeStruct((B, L, OUT_D), jnp.float32),
        mesh=mesh,
        compiler_params=pltpu.CompilerParams(needs_layout_passes=False,
                                             use_tc_tiling_on_sc=False),
        scratch_types=[
            pltpu.VMEM((2, E_TILE, OUT_D), jnp.float32),
            pltpu.VMEM((2, E_TILE, G_D), jnp.float32),
            pltpu.VMEM_SHARED((NS, E_TILE, G_D), jnp.float32),
            pltpu.VMEM((2, E_TILE), jnp.int32),
            pltpu.VMEM((2, E_TILE), jnp.int32),
            pltpu.VMEM((2, E_TILE), jnp.int32),
            pltpu.VMEM((2, E_TILE), jnp.int32),
            pltpu.VMEM((2, E_TILE), jnp.int32),
            pltpu.VMEM((2, E_TILE), jnp.int32),
            pltpu.VMEM((2, E_TILE), jnp.int32),
            pltpu.VMEM((2, E_TILE * 4), jnp.float32),
            pltpu.VMEM((16, 4), jnp.float32),
            pltpu.VMEM((8, 4), jnp.float32),
            pltpu.VMEM((8, 4), jnp.float32),
            pltpu.VMEM((16, 4), jnp.float32),
            pltpu.VMEM((16,), jnp.float32),
            pltpu.VMEM((16,), jnp.float32),
            pltpu.VMEM((R_TILE, 16), jnp.float32),
            pltpu.VMEM((R_TILE, 16), jnp.float32),
            pltpu.SemaphoreType.DMA,
            pltpu.SemaphoreType.DMA,
            pltpu.SemaphoreType.DMA,
        ],
    )
    return f(tpad, idx_f, enh_f, edi_f, seal_f, seg_f, suit_f, rank_f,
             scal_f, enh_table, edi_table, seal_table, seg_table,
             sin_tab, cos_tab)


def kernel(indices, enhancement, edition, seal, segment, suit, rank,
           scalar_properties, debuffed,
           idx_table, enh_table, edi_table, seal_table, seg_table,
           suit_table, rank_table):
    tpad = jnp.pad(idx_table, ((0, 0), (0, G_D - idx_table.shape[1])))
    ang = jnp.arange(16, dtype=jnp.float32) * (2.0 * math.pi / 13.0)
    sin_tab = jnp.sin(ang)
    cos_tab = jnp.cos(ang)
    scale = jnp.array([10.0, 100.0, 100.0, 10.0], jnp.float32)
    scal_pre = (scalar_properties
                * (1.0 - debuffed.astype(jnp.float32))[:, :, None] / scale)
    emb = _run(
        tpad,
        indices.reshape(-1), enhancement.reshape(-1), edition.reshape(-1),
        seal.reshape(-1), segment.reshape(-1), suit.reshape(-1),
        rank.reshape(-1),
        scal_pre.reshape(-1),
        enh_table, edi_table, seal_table, seg_table, sin_tab, cos_tab)
    return emb, (indices == 0) & (rank == 0)


# speedup vs baseline: 1.1578x; 1.0432x over previous
import math

import jax
import jax.numpy as jnp
from jax import lax
from jax.experimental import pallas as pl
from jax.experimental.pallas import tpu as pltpu
from jax.experimental.pallas import tpu_sc as plsc

B, L = 4096, 50
N = B * L
OUT_D = 63
G_D = 48
NC, NS = 2, 16
NW = NC * NS
ROWS_PER_W = B // NW
R_TILE = 8
E_TILE = R_TILE * L
N_TILES = ROWS_PER_W // R_TILE
G_CHUNKS = [(lo, min(128, E_TILE - lo)) for lo in range(0, E_TILE, 128)]


def _body(tpad_h, idx_h, pk_h,
          scal_h, enht_h, edit_h, sealt_h, segt_h, sin_h, cos_h,
          out_h,
          out_t, g48, sp48, idx_v, pk_v, scal_v, enh_t, edi_t, seal_t,
          seg_t, sin_t, cos_t, rhist, shist, sem_in, sem_g, sem_out):
    wid = lax.axis_index("s") * NC + lax.axis_index("c")
    sid = lax.axis_index("s")
    pltpu.sync_copy(enht_h, enh_t)
    pltpu.sync_copy(edit_h, edi_t)
    pltpu.sync_copy(sealt_h, seal_t)
    pltpu.sync_copy(segt_h, seg_t)
    pltpu.sync_copy(sin_h, sin_t)
    pltpu.sync_copy(cos_h, cos_t)

    iota = lax.iota(jnp.int32, 16)
    ones = jnp.ones((16,), jnp.float32)
    base_w = wid * (ROWS_PER_W * L)
    rowb_w = wid * ROWS_PER_W

    def col(c):
        return jnp.full((16,), c, jnp.int32)

    def fire_in(t, b):
        base = base_w + t * E_TILE
        rowb = rowb_w + t * R_TILE
        pltpu.async_copy(idx_h.at[pl.ds(base, E_TILE)], idx_v.at[b], sem_in)
        pltpu.async_copy(pk_h.at[pl.ds(base, E_TILE)], pk_v.at[b], sem_in)
        pltpu.async_copy(scal_h.at[pl.ds(base * 4, E_TILE * 4)], scal_v.at[b],
                         sem_in)

    def drain_in():
        pltpu.make_async_copy(idx_h.at[pl.ds(0, E_TILE)], idx_v.at[0],
                              sem_in).wait()
        pltpu.make_async_copy(pk_h.at[pl.ds(0, E_TILE)], pk_v.at[0],
                              sem_in).wait()
        pltpu.make_async_copy(scal_h.at[pl.ds(0, E_TILE * 4)], scal_v.at[0],
                              sem_in).wait()

    def fire_g(b):
        for lo, n in G_CHUNKS:
            pltpu.async_copy(tpad_h.at[idx_v.at[b, pl.ds(lo, n)]],
                             g48.at[b, pl.ds(lo, n)], sem_g)

    def drain_g():
        for lo, n in G_CHUNKS:
            pltpu.make_async_copy(tpad_h.at[pl.ds(0, n)],
                                  g48.at[0, pl.ds(0, n)], sem_g).wait()

    def fire_out(t, b):
        rowb = rowb_w + t * R_TILE
        for r in range(R_TILE):
            pltpu.async_copy(out_t.at[b, pl.ds(L * r, L)],
                             out_h.at[rowb + r], sem_out)


    def drain_out():
        for r in range(R_TILE):
            pltpu.make_async_copy(out_h.at[0], out_t.at[0, pl.ds(0, L)],
                                  sem_out).wait()


    def assemble(b):
        bb = jnp.full((16,), b, jnp.int32)

        @plsc.parallel_loop(0, R_TILE, step=1, carry=jnp.int32(0))
        def _rows(r, cry):
            e0 = r * L
            rr = iota * 0 + r
            zf = jnp.zeros((16,), jnp.float32)
            plsc.store_scatter(rhist, [rr, iota], zf)
            plsc.store_scatter(shist, [rr, iota], zf)
            groups = []
            for g in range(4):
                rem = min(16, L - 16 * g)
                valid = iota < rem
                lidx = jnp.minimum(16 * g + iota, L - 1)
                eidx = e0 + lidx
                pkv = plsc.load_gather(pk_v, [bb, eidx])
                rv = jnp.where(valid, (pkv >> 17) & 15, 15)
                sv = jnp.where(valid, (pkv >> 14) & 7, 15)
                groups.append((lidx, eidx, valid, rv, sv, pkv))
                plsc.addupdate_scatter(rhist, [rr, rv], ones, mask=valid)
                plsc.addupdate_scatter(shist, [rr, sv], ones, mask=valid)
            for lidx, eidx, valid, rv, sv, pkv in groups:
                same_rank = plsc.load_gather(rhist, [rr, rv])
                same_suit = plsc.load_gather(shist, [rr, sv])
                up = plsc.load_gather(rhist, [rr, jnp.maximum(rv - 1, 0)])
                down = plsc.load_gather(rhist, [rr, jnp.minimum(rv + 1, 15)])
                m0 = rv == 0
                zero = jnp.zeros((16,), jnp.float32)
                f_sr = jnp.where(m0, zero, same_rank * 0.2)
                f_ss = jnp.where(m0, zero, same_suit * 0.2)
                f_up = jnp.where(m0, zero, up)
                f_dn = jnp.where(m0, zero, down)
                sinv = plsc.load_gather(sin_t, [rv])
                cosv = plsc.load_gather(cos_t, [rv])
                plsc.addupdate_scatter(out_t, [bb, eidx, col(37)], cosv,
                                       mask=valid)
                plsc.addupdate_scatter(out_t, [bb, eidx, col(38)], sinv,
                                       mask=valid)
                plsc.addupdate_scatter(out_t, [bb, eidx, col(39)], f_dn,
                                       mask=valid)
                plsc.addupdate_scatter(out_t, [bb, eidx, col(40)], f_up,
                                       mask=valid)
                plsc.addupdate_scatter(out_t, [bb, eidx, col(41)], f_ss,
                                       mask=valid)
                plsc.addupdate_scatter(out_t, [bb, eidx, col(42)], f_sr,
                                       mask=valid)
                plsc.addupdate_scatter(out_t, [bb, eidx, sv], ones,
                                       mask=valid)
                plsc.addupdate_scatter(out_t, [bb, eidx, rv + 5], ones,
                                       mask=valid)
                for fv, tab_ref, c0 in (((pkv >> 10) & 15, seg_t, 47),
                                        (pkv & 15, enh_t, 51),
                                        ((pkv >> 4) & 7, edi_t, 55),
                                        ((pkv >> 7) & 7, seal_t, 59)):
                    for c in range(4):
                        plsc.store_scatter(
                            out_t, [bb, eidx, col(c0 + c)],
                            plsc.load_gather(tab_ref, [fv, col(c)]),
                            mask=valid)
                for c in range(4):
                    s = plsc.load_gather(scal_v, [bb, eidx * 4 + c])
                    plsc.store_scatter(out_t, [bb, eidx, col(43 + c)], s,
                                       mask=valid)
            return cry

    fire_in(0, 0)
    drain_in()
    fire_g(0)
    fire_in(1, 1)

    def tile_body(t, carry):
        b = t % 2
        drain_g()
        pltpu.sync_copy(g48.at[b], sp48.at[sid])
        pltpu.sync_copy(sp48.at[sid], out_t.at[b, :, pl.ds(0, G_D)])

        @pl.when(t < N_TILES - 1)
        def _():
            drain_in()

        assemble(b)

        @pl.when(t > 0)
        def _():
            drain_out()
        fire_out(t, b)

        @pl.when(t < N_TILES - 1)
        def _():
            fire_g(1 - b)

        @pl.when(t < N_TILES - 2)
        def _():
            fire_in(jnp.minimum(t + 2, N_TILES - 1), b)
        return carry

    lax.fori_loop(0, N_TILES, tile_body, 0)
    drain_out()


@jax.jit
def _run(tpad, idx_f, pk_f,
         scal_f, enh_table, edi_table, seal_table, seg_table, sin_tab,
         cos_tab):
    mesh = plsc.VectorSubcoreMesh(core_axis_name="c", subcore_axis_name="s")
    f = pl.kernel(
        _body,
        out_type=jax.ShapeDtypeStruct((B, L, OUT_D), jnp.float32),
        mesh=mesh,
        compiler_params=pltpu.CompilerParams(needs_layout_passes=False,
                                             use_tc_tiling_on_sc=False),
        scratch_types=[
            pltpu.VMEM((2, E_TILE, OUT_D), jnp.float32),
            pltpu.VMEM((2, E_TILE, G_D), jnp.float32),
            pltpu.VMEM_SHARED((NS, E_TILE, G_D), jnp.float32),
            pltpu.VMEM((2, E_TILE), jnp.int32),
            pltpu.VMEM((2, E_TILE), jnp.int32),
            pltpu.VMEM((2, E_TILE * 4), jnp.float32),
            pltpu.VMEM((16, 4), jnp.float32),
            pltpu.VMEM((8, 4), jnp.float32),
            pltpu.VMEM((8, 4), jnp.float32),
            pltpu.VMEM((16, 4), jnp.float32),
            pltpu.VMEM((16,), jnp.float32),
            pltpu.VMEM((16,), jnp.float32),
            pltpu.VMEM((R_TILE, 16), jnp.float32),
            pltpu.VMEM((R_TILE, 16), jnp.float32),
            pltpu.SemaphoreType.DMA,
            pltpu.SemaphoreType.DMA,
            pltpu.SemaphoreType.DMA,
        ],
    )
    return f(tpad, idx_f, pk_f,
             scal_f, enh_table, edi_table, seal_table, seg_table,
             sin_tab, cos_tab)


def kernel(indices, enhancement, edition, seal, segment, suit, rank,
           scalar_properties, debuffed,
           idx_table, enh_table, edi_table, seal_table, seg_table,
           suit_table, rank_table):
    tpad = jnp.pad(idx_table, ((0, 0), (0, G_D - idx_table.shape[1])))
    ang = jnp.arange(16, dtype=jnp.float32) * (2.0 * math.pi / 13.0)
    sin_tab = jnp.sin(ang)
    cos_tab = jnp.cos(ang)
    scale = jnp.array([10.0, 100.0, 100.0, 10.0], jnp.float32)
    scal_pre = (scalar_properties
                * (1.0 - debuffed.astype(jnp.float32))[:, :, None] / scale)
    pk = (enhancement | (edition << 4) | (seal << 7) | (segment << 10)
          | (suit << 14) | (rank << 17))
    emb = _run(
        tpad,
        indices.reshape(-1), pk.reshape(-1),
        scal_pre.reshape(-1),
        enh_table, edi_table, seal_table, seg_table, sin_tab, cos_tab)
    return emb, (indices == 0) & (rank == 0)


# speedup vs baseline: 1.4486x; 1.2511x over previous
import math

import jax
import jax.numpy as jnp
from jax import lax
from jax.experimental import pallas as pl
from jax.experimental.pallas import tpu as pltpu
from jax.experimental.pallas import tpu_sc as plsc

B, L = 4096, 50
N = B * L
OUT_D = 63
G_D = 48
NC, NS = 2, 16
NW = NC * NS
ROWS_PER_W = B // NW
R_TILE = 8
E_TILE = R_TILE * L
N_TILES = ROWS_PER_W // R_TILE
G_CHUNKS = [(lo, min(128, E_TILE - lo)) for lo in range(0, E_TILE, 128)]


def _body(tpad_h, idx_h, pk_h,
          scal_h, enht_h, edit_h, sealt_h, segt_h, sin_h, cos_h,
          out_h,
          out_t, g48, sp48, idx_v, pk_v, scal_v, enh_t, edi_t, seal_t,
          seg_t, sin_t, cos_t, rhist, shist, sem_in, sem_g, sem_out):
    wid = lax.axis_index("s") * NC + lax.axis_index("c")
    sid = lax.axis_index("s")
    pltpu.sync_copy(enht_h, enh_t)
    pltpu.sync_copy(edit_h, edi_t)
    pltpu.sync_copy(sealt_h, seal_t)
    pltpu.sync_copy(segt_h, seg_t)
    pltpu.sync_copy(sin_h, sin_t)
    pltpu.sync_copy(cos_h, cos_t)

    iota = lax.iota(jnp.int32, 16)
    ones = jnp.ones((16,), jnp.float32)
    base_w = wid * (ROWS_PER_W * L)
    rowb_w = wid * ROWS_PER_W

    def col(c):
        return jnp.full((16,), c, jnp.int32)

    def fire_in(t, b):
        base = base_w + t * E_TILE
        rowb = rowb_w + t * R_TILE
        pltpu.async_copy(idx_h.at[pl.ds(base, E_TILE)], idx_v.at[b], sem_in)
        pltpu.async_copy(pk_h.at[pl.ds(base, E_TILE)], pk_v.at[b], sem_in)
        pltpu.async_copy(scal_h.at[pl.ds(rowb, R_TILE)], scal_v.at[b],
                         sem_in)

    def drain_in():
        pltpu.make_async_copy(idx_h.at[pl.ds(0, E_TILE)], idx_v.at[0],
                              sem_in).wait()
        pltpu.make_async_copy(pk_h.at[pl.ds(0, E_TILE)], pk_v.at[0],
                              sem_in).wait()
        pltpu.make_async_copy(scal_h.at[pl.ds(0, R_TILE)], scal_v.at[0],
                              sem_in).wait()

    def fire_g(b):
        for lo, n in G_CHUNKS:
            pltpu.async_copy(tpad_h.at[idx_v.at[b, pl.ds(lo, n)]],
                             g48.at[b, pl.ds(lo, n)], sem_g)

    def drain_g():
        for lo, n in G_CHUNKS:
            pltpu.make_async_copy(tpad_h.at[pl.ds(0, n)],
                                  g48.at[0, pl.ds(0, n)], sem_g).wait()

    def fire_out(t, b):
        rowb = rowb_w + t * R_TILE
        for r in range(R_TILE):
            pltpu.async_copy(out_t.at[b, pl.ds(L * r, L)],
                             out_h.at[rowb + r], sem_out)


    def drain_out():
        for r in range(R_TILE):
            pltpu.make_async_copy(out_h.at[0], out_t.at[0, pl.ds(0, L)],
                                  sem_out).wait()


    def assemble(b):
        bb = jnp.full((16,), b, jnp.int32)

        @plsc.parallel_loop(0, R_TILE, step=1, carry=jnp.int32(0))
        def _rows(r, cry):
            e0 = r * L
            rr = iota * 0 + r
            zf = jnp.zeros((16,), jnp.float32)
            plsc.store_scatter(rhist, [rr, iota], zf)
            plsc.store_scatter(shist, [rr, iota], zf)
            groups = []
            for g in range(4):
                rem = min(16, L - 16 * g)
                valid = iota < rem
                lidx = jnp.minimum(16 * g + iota, L - 1)
                eidx = e0 + lidx
                pkv = plsc.load_gather(pk_v, [bb, eidx])
                rv = jnp.where(valid, (pkv >> 17) & 15, 15)
                sv = jnp.where(valid, (pkv >> 14) & 7, 15)
                groups.append((lidx, eidx, valid, rv, sv, pkv))
                plsc.addupdate_scatter(rhist, [rr, rv], ones, mask=valid)
                plsc.addupdate_scatter(shist, [rr, sv], ones, mask=valid)
            for lidx, eidx, valid, rv, sv, pkv in groups:
                same_rank = plsc.load_gather(rhist, [rr, rv])
                same_suit = plsc.load_gather(shist, [rr, sv])
                up = plsc.load_gather(rhist, [rr, jnp.maximum(rv - 1, 0)])
                down = plsc.load_gather(rhist, [rr, jnp.minimum(rv + 1, 15)])
                m0 = rv == 0
                zero = jnp.zeros((16,), jnp.float32)
                f_sr = jnp.where(m0, zero, same_rank * 0.2)
                f_ss = jnp.where(m0, zero, same_suit * 0.2)
                f_up = jnp.where(m0, zero, up)
                f_dn = jnp.where(m0, zero, down)
                sinv = plsc.load_gather(sin_t, [rv])
                cosv = plsc.load_gather(cos_t, [rv])
                plsc.addupdate_scatter(out_t, [bb, eidx, col(37)], cosv,
                                       mask=valid)
                plsc.addupdate_scatter(out_t, [bb, eidx, col(38)], sinv,
                                       mask=valid)
                plsc.addupdate_scatter(out_t, [bb, eidx, col(39)], f_dn,
                                       mask=valid)
                plsc.addupdate_scatter(out_t, [bb, eidx, col(40)], f_up,
                                       mask=valid)
                plsc.addupdate_scatter(out_t, [bb, eidx, col(41)], f_ss,
                                       mask=valid)
                plsc.addupdate_scatter(out_t, [bb, eidx, col(42)], f_sr,
                                       mask=valid)
                plsc.addupdate_scatter(out_t, [bb, eidx, sv], ones,
                                       mask=valid)
                plsc.addupdate_scatter(out_t, [bb, eidx, rv + 5], ones,
                                       mask=valid)
                for fv, tab_ref, c0 in (((pkv >> 10) & 15, seg_t, 47),
                                        (pkv & 15, enh_t, 51),
                                        ((pkv >> 4) & 7, edi_t, 55),
                                        ((pkv >> 7) & 7, seal_t, 59)):
                    for c in range(4):
                        plsc.store_scatter(
                            out_t, [bb, eidx, col(c0 + c)],
                            plsc.load_gather(tab_ref, [fv, col(c)]),
                            mask=valid)
                for c in range(4):
                    s = plsc.load_gather(scal_v, [bb, rr, lidx * 4 + c])
                    plsc.store_scatter(out_t, [bb, eidx, col(43 + c)], s,
                                       mask=valid)
            return cry

    fire_in(0, 0)
    drain_in()
    fire_g(0)
    fire_in(1, 1)

    def tile_body(t, carry):
        b = t % 2
        drain_g()
        pltpu.sync_copy(g48.at[b], sp48.at[sid])
        pltpu.sync_copy(sp48.at[sid], out_t.at[b, :, pl.ds(0, G_D)])

        @pl.when(t < N_TILES - 1)
        def _():
            drain_in()

        assemble(b)

        @pl.when(t > 0)
        def _():
            drain_out()
        fire_out(t, b)

        @pl.when(t < N_TILES - 1)
        def _():
            fire_g(1 - b)

        @pl.when(t < N_TILES - 2)
        def _():
            fire_in(jnp.minimum(t + 2, N_TILES - 1), b)
        return carry

    lax.fori_loop(0, N_TILES, tile_body, 0)
    drain_out()


@jax.jit
def _run(tpad, idx_f, pk_f,
         scal_f, enh_table, edi_table, seal_table, seg_table, sin_tab,
         cos_tab):
    mesh = plsc.VectorSubcoreMesh(core_axis_name="c", subcore_axis_name="s")
    f = pl.kernel(
        _body,
        out_type=jax.ShapeDtypeStruct((B, L, OUT_D), jnp.float32),
        mesh=mesh,
        compiler_params=pltpu.CompilerParams(needs_layout_passes=False,
                                             use_tc_tiling_on_sc=False),
        scratch_types=[
            pltpu.VMEM((2, E_TILE, OUT_D), jnp.float32),
            pltpu.VMEM((2, E_TILE, G_D), jnp.float32),
            pltpu.VMEM_SHARED((NS, E_TILE, G_D), jnp.float32),
            pltpu.VMEM((2, E_TILE), jnp.int32),
            pltpu.VMEM((2, E_TILE), jnp.int32),
            pltpu.VMEM((2, R_TILE, 4 * L), jnp.float32),
            pltpu.VMEM((16, 4), jnp.float32),
            pltpu.VMEM((8, 4), jnp.float32),
            pltpu.VMEM((8, 4), jnp.float32),
            pltpu.VMEM((16, 4), jnp.float32),
            pltpu.VMEM((16,), jnp.float32),
            pltpu.VMEM((16,), jnp.float32),
            pltpu.VMEM((R_TILE, 16), jnp.float32),
            pltpu.VMEM((R_TILE, 16), jnp.float32),
            pltpu.SemaphoreType.DMA,
            pltpu.SemaphoreType.DMA,
            pltpu.SemaphoreType.DMA,
        ],
    )
    return f(tpad, idx_f, pk_f,
             scal_f, enh_table, edi_table, seal_table, seg_table,
             sin_tab, cos_tab)


def kernel(indices, enhancement, edition, seal, segment, suit, rank,
           scalar_properties, debuffed,
           idx_table, enh_table, edi_table, seal_table, seg_table,
           suit_table, rank_table):
    tpad = jnp.pad(idx_table, ((0, 0), (0, G_D - idx_table.shape[1])))
    ang = jnp.arange(16, dtype=jnp.float32) * (2.0 * math.pi / 13.0)
    sin_tab = jnp.sin(ang)
    cos_tab = jnp.cos(ang)
    scale = jnp.array([10.0, 100.0, 100.0, 10.0], jnp.float32)
    scal_pre = (scalar_properties
                * (1.0 - debuffed.astype(jnp.float32))[:, :, None] / scale)
    pk = (enhancement | (edition << 4) | (seal << 7) | (segment << 10)
          | (suit << 14) | (rank << 17))
    emb = _run(
        tpad,
        indices.reshape(-1), pk.reshape(-1),
        scal_pre.reshape(B, 4 * L),
        enh_table, edi_table, seal_table, seg_table, sin_tab, cos_tab)
    return emb, (indices == 0) & (rank == 0)
